# baseline (speedup 1.0000x reference)
import math

import jax
import jax.numpy as jnp
from jax import lax
from jax.experimental import pallas as pl
from jax.experimental.pallas import tpu as pltpu

N = 524288
C = 32
R = 4096
L = 128
LOGN = 19
LOGR = 12

_row_j, _lane_j, _row_m, _lane_m = [], [], [], []
for _k in range(1, LOGN + 1):
    for _jexp in range(_k - 1, -1, -1):
        _j = 1 << _jexp
        if _jexp < LOGR:
            _row_j.append(_j)
            _lane_j.append(0)
        else:
            _row_j.append(0)
            _lane_j.append(_j >> LOGR)
        if _k < LOGR:
            _row_m.append(1 << _k)
            _lane_m.append(0)
        else:
            _row_m.append(0)
            _lane_m.append((1 << _k) >> LOGR)
S = len(_row_j)


def _body(rj_ref, lj_ref, rm_ref, lm_ref, x_ref, g_ref, out_ref, a_ref, b_ref):
    col = pl.program_id(0)
    s = pl.program_id(1)

    @pl.when(jnp.logical_and(col == 0, s == 0))
    def _():
        out_ref[0, 0] = 0.0

    @pl.when(s == 0)
    def _():
        a_ref[...] = x_ref[0]
        b_ref[...] = g_ref[0]

    rj = rj_ref[s]
    lj = lj_ref[s]
    rm = rm_ref[s]
    lm = lm_ref[s]
    r_iota = lax.broadcasted_iota(jnp.int32, (R, L), 0)
    c_iota = lax.broadcasted_iota(jnp.int32, (R, L), 1)
    bit = ((r_iota & rj) | (c_iota & lj)) != 0
    desc = ((r_iota & rm) | (c_iota & lm)) != 0
    want_min = bit == desc

    a = a_ref[...]
    b = b_ref[...]

    def lane_branch(a, b):
        return (pltpu.roll(a, L - lj, 1), pltpu.roll(a, lj, 1),
                pltpu.roll(b, L - lj, 1), pltpu.roll(b, lj, 1))

    def row_branch(a, b):
        return (pltpu.roll(a, R - rj, 0), pltpu.roll(a, rj, 0),
                pltpu.roll(b, R - rj, 0), pltpu.roll(b, rj, 0))

    ua, da, ub, db = lax.cond(lj > 0, lane_branch, row_branch, a, b)
    pa = jnp.where(bit, da, ua)
    pb = jnp.where(bit, db, ub)
    a_ref[...] = jnp.where(want_min, jnp.minimum(a, pa), jnp.maximum(a, pa))
    b_ref[...] = jnp.where(want_min, jnp.minimum(b, pb), jnp.maximum(b, pb))

    @pl.when(s == S - 1)
    def _():
        x = b_ref[...] - a_ref[...]
        loss = jnp.maximum(x, 0.0) - x + jnp.log1p(jnp.exp(-jnp.abs(x)))
        out_ref[0, 0] += jnp.sum(loss)


def kernel(true_data, fake_data):
    rj_a = jnp.array(_row_j, dtype=jnp.int32)
    lj_a = jnp.array(_lane_j, dtype=jnp.int32)
    rm_a = jnp.array(_row_m, dtype=jnp.int32)
    lm_a = jnp.array(_lane_m, dtype=jnp.int32)
    tx = jnp.transpose(true_data.reshape(L, R, C), (2, 1, 0))
    tg = jnp.transpose(fake_data.reshape(L, R, C), (2, 1, 0))
    total = pl.pallas_call(
        _body,
        grid=(C, S),
        in_specs=[
            pl.BlockSpec(memory_space=pltpu.SMEM),
            pl.BlockSpec(memory_space=pltpu.SMEM),
            pl.BlockSpec(memory_space=pltpu.SMEM),
            pl.BlockSpec(memory_space=pltpu.SMEM),
            pl.BlockSpec((1, R, L), lambda col, s: (col, 0, 0)),
            pl.BlockSpec((1, R, L), lambda col, s: (col, 0, 0)),
        ],
        out_specs=pl.BlockSpec(memory_space=pltpu.SMEM),
        out_shape=jax.ShapeDtypeStruct((1, 1), jnp.float32),
        scratch_shapes=[
            pltpu.VMEM((R, L), jnp.float32),
            pltpu.VMEM((R, L), jnp.float32),
        ],
    )(rj_a, lj_a, rm_a, lm_a, tx, tg)
    return total[0, 0] / (N * C)
